# baseline (speedup 1.0000x reference)
import functools

import jax
import jax.numpy as jnp
from jax import lax
from jax.experimental import pallas as pl
from jax.experimental.pallas import tpu as pltpu
from jax.experimental.pallas import tpu_sc as plsc

_TILE = 512
_NC = 2
_NS = 16
_LANES = 16


def _stage_a1(xt_ref, y_ref, idx_ref, w_ref, *, S):
    xt = xt_ref[0]
    y = y_ref[0]
    xx = jnp.sum(xt * xt, axis=0, keepdims=True)
    yy = jnp.sum(y * y, axis=1, keepdims=True)
    dt = yy - 2.0 * jnp.dot(y, xt, preferred_element_type=jnp.float32) + xx

    ii = lax.broadcasted_iota(jnp.int32, dt.shape, 0)
    BIG = jnp.float32(3.0e38)
    m1 = jnp.min(dt, axis=0, keepdims=True)
    i1 = jnp.min(jnp.where(dt == m1, ii, S), axis=0, keepdims=True)
    d2 = jnp.where(ii == i1, BIG, dt)
    m2 = jnp.min(d2, axis=0, keepdims=True)
    i2 = jnp.min(jnp.where(d2 == m2, ii, S), axis=0, keepdims=True)
    d3 = jnp.where(ii == i2, BIG, d2)
    m3 = jnp.min(d3, axis=0, keepdims=True)
    i3 = jnp.min(jnp.where(d3 == m3, ii, S), axis=0, keepdims=True)

    mm = jnp.concatenate([m1, m2, m3], axis=0)
    r = 1.0 / (mm + 1e-8)
    w_ref[...] = r / jnp.sum(r, axis=0, keepdims=True)
    idx_ref[...] = (jnp.concatenate([i1, i2, i3], axis=0)
                    + pl.program_id(0) * S)


def _sc_interp(i1, i2, i3, w1, w2, w3, table, BN, D2):
    NW = _NC * _NS
    QW = BN // NW
    Q = 16
    NB = QW // Q
    DCH = D2 // _LANES

    mesh = plsc.VectorSubcoreMesh(core_axis_name="c", subcore_axis_name="s")

    @functools.partial(
        pl.kernel,
        mesh=mesh,
        out_type=jax.ShapeDtypeStruct((BN, D2), jnp.float32),
        scratch_types=[
            pltpu.VMEM((QW,), jnp.int32),
            pltpu.VMEM((QW,), jnp.int32),
            pltpu.VMEM((QW,), jnp.int32),
            pltpu.VMEM((QW,), jnp.float32),
            pltpu.VMEM((QW,), jnp.float32),
            pltpu.VMEM((QW,), jnp.float32),
            pltpu.VMEM((Q, D2), jnp.float32),
            pltpu.VMEM((Q, D2), jnp.float32),
            pltpu.VMEM((Q, D2), jnp.float32),
            pltpu.VMEM((Q, D2), jnp.float32),
            pltpu.SemaphoreType.DMA,
            pltpu.SemaphoreType.DMA,
            pltpu.SemaphoreType.DMA,
        ],
    )
    def gather_interp(i1_hbm, i2_hbm, i3_hbm, w1_hbm, w2_hbm, w3_hbm,
                      table_hbm, out_hbm,
                      i1v, i2v, i3v, w1v, w2v, w3v,
                      r1v, r2v, r3v, outv, sem1, sem2, sem3):
        wid = lax.axis_index("s") * _NC + lax.axis_index("c")
        base = wid * QW
        pltpu.sync_copy(i1_hbm.at[pl.ds(base, QW)], i1v)
        pltpu.sync_copy(i2_hbm.at[pl.ds(base, QW)], i2v)
        pltpu.sync_copy(i3_hbm.at[pl.ds(base, QW)], i3v)
        pltpu.sync_copy(w1_hbm.at[pl.ds(base, QW)], w1v)
        pltpu.sync_copy(w2_hbm.at[pl.ds(base, QW)], w2v)
        pltpu.sync_copy(w3_hbm.at[pl.ds(base, QW)], w3v)

        def block(j, carry):
            qb = pl.multiple_of(j * Q, Q)
            cp1 = pltpu.async_copy(table_hbm.at[i1v[pl.ds(qb, Q)]], r1v, sem1)
            cp2 = pltpu.async_copy(table_hbm.at[i2v[pl.ds(qb, Q)]], r2v, sem2)
            cp3 = pltpu.async_copy(table_hbm.at[i3v[pl.ds(qb, Q)]], r3v, sem3)
            wa = w1v[pl.ds(qb, Q)]
            wb = w2v[pl.ds(qb, Q)]
            wc = w3v[pl.ds(qb, Q)]
            cp1.wait()
            cp2.wait()
            cp3.wait()
            for q in range(Q):
                qi = jnp.full((_LANES,), q, jnp.int32)
                ba = wa.at[qi].get(mode="promise_in_bounds")
                bb = wb.at[qi].get(mode="promise_in_bounds")
                bc = wc.at[qi].get(mode="promise_in_bounds")
                for d in range(DCH):
                    sl = pl.ds(d * _LANES, _LANES)
                    outv[q, sl] = (ba * r1v[q, sl] + bb * r2v[q, sl]
                                   + bc * r3v[q, sl])
            pltpu.sync_copy(outv, out_hbm.at[pl.ds(base + qb, Q)])
            return carry

        lax.fori_loop(0, NB, block, 0)

    return gather_interp(i1, i2, i3, w1, w2, w3, table)


def _stage_a2(p1_ref, it_ref, w0t_ref, b0_ref, y0_ref, s0_ref, q0_ref):
    cat = jnp.concatenate([p1_ref[0], it_ref[0]], axis=1)
    y0 = jnp.dot(cat, w0t_ref[...], preferred_element_type=jnp.float32) + b0_ref[...]
    y0_ref[0] = y0

    @pl.when((pl.program_id(0) == 0) & (pl.program_id(1) == 0))
    def _():
        s0_ref[...] = jnp.zeros_like(s0_ref)
        q0_ref[...] = jnp.zeros_like(q0_ref)

    s0_ref[...] += jnp.sum(y0, axis=0, keepdims=True)
    q0_ref[...] += jnp.sum(y0 * y0, axis=0, keepdims=True)


def _stage_b(y0_ref, s0_ref, q0_ref, g0_ref, be0_ref, w1t_ref, b1_ref,
             y1_ref, s1_ref, q1_ref, *, inv_m):
    mean = s0_ref[...] * inv_m
    var = q0_ref[...] * inv_m - mean * mean
    scale = g0_ref[...] * lax.rsqrt(var + 1e-5)
    shift = be0_ref[...] - mean * scale
    x1 = jnp.maximum(y0_ref[0] * scale + shift, 0.0)
    y1 = jnp.dot(x1, w1t_ref[...], preferred_element_type=jnp.float32) + b1_ref[...]
    y1_ref[0] = y1

    @pl.when((pl.program_id(0) == 0) & (pl.program_id(1) == 0))
    def _():
        s1_ref[...] = jnp.zeros_like(s1_ref)
        q1_ref[...] = jnp.zeros_like(q1_ref)

    s1_ref[...] += jnp.sum(y1, axis=0, keepdims=True)
    q1_ref[...] += jnp.sum(y1 * y1, axis=0, keepdims=True)


def _stage_c(y1_ref, s1_ref, q1_ref, g1_ref, be1_ref, out_ref, *, inv_m):
    mean = s1_ref[...] * inv_m
    var = q1_ref[...] * inv_m - mean * mean
    scale = g1_ref[...] * lax.rsqrt(var + 1e-5)
    shift = be1_ref[...] - mean * scale
    x2 = jnp.maximum(y1_ref[0] * scale + shift, 0.0)
    out_ref[0] = x2.T


def kernel(xyz1, xyz2, points1, points2, W0, b0, g0, be0, W1, b1, g1, be1):
    B, N, _ = xyz1.shape
    S = xyz2.shape[2]
    D1 = points1.shape[2]
    D2 = points2.shape[1]
    C0 = W0.shape[0]
    C1 = W1.shape[0]
    T = _TILE
    NT = N // T
    BN = B * N
    inv_m = 1.0 / float(BN)

    xyz1t = jnp.transpose(xyz1, (0, 2, 1))
    xyz2t = jnp.transpose(xyz2, (0, 2, 1))
    table = jnp.transpose(points2, (0, 2, 1)).reshape(B * S, D2)
    w0t = W0.T
    w1t = W1.T
    b0r, g0r, be0r = b0.reshape(1, C0), g0.reshape(1, C0), be0.reshape(1, C0)
    b1r, g1r, be1r = b1.reshape(1, C1), g1.reshape(1, C1), be1.reshape(1, C1)

    stats_spec_c0 = pl.BlockSpec((1, C0), lambda b, n: (0, 0))
    stats_spec_c1 = pl.BlockSpec((1, C1), lambda b, n: (0, 0))
    params = pltpu.CompilerParams(dimension_semantics=("arbitrary", "arbitrary"))

    idx3, w3 = pl.pallas_call(
        functools.partial(_stage_a1, S=S),
        grid=(B, NT),
        in_specs=[
            pl.BlockSpec((1, 3, T), lambda b, n: (b, 0, n)),
            pl.BlockSpec((1, S, 3), lambda b, n: (b, 0, 0)),
        ],
        out_specs=[
            pl.BlockSpec((3, T), lambda b, n, NT=NT: (0, b * NT + n)),
            pl.BlockSpec((3, T), lambda b, n, NT=NT: (0, b * NT + n)),
        ],
        out_shape=[
            jax.ShapeDtypeStruct((3, BN), jnp.int32),
            jax.ShapeDtypeStruct((3, BN), jnp.float32),
        ],
        compiler_params=params,
    )(xyz1t, xyz2t)

    interp = _sc_interp(idx3[0], idx3[1], idx3[2], w3[0], w3[1], w3[2],
                        table, BN, D2)
    interp = interp.reshape(B, N, D2)

    y0, s0, q0 = pl.pallas_call(
        _stage_a2,
        grid=(B, NT),
        in_specs=[
            pl.BlockSpec((1, T, D1), lambda b, n: (b, n, 0)),
            pl.BlockSpec((1, T, D2), lambda b, n: (b, n, 0)),
            pl.BlockSpec((D1 + D2, C0), lambda b, n: (0, 0)),
            stats_spec_c0,
        ],
        out_specs=[
            pl.BlockSpec((1, T, C0), lambda b, n: (b, n, 0)),
            stats_spec_c0,
            stats_spec_c0,
        ],
        out_shape=[
            jax.ShapeDtypeStruct((B, N, C0), jnp.float32),
            jax.ShapeDtypeStruct((1, C0), jnp.float32),
            jax.ShapeDtypeStruct((1, C0), jnp.float32),
        ],
        compiler_params=params,
    )(points1, interp, w0t, b0r)

    y1, s1, q1 = pl.pallas_call(
        functools.partial(_stage_b, inv_m=inv_m),
        grid=(B, NT),
        in_specs=[
            pl.BlockSpec((1, T, C0), lambda b, n: (b, n, 0)),
            stats_spec_c0,
            stats_spec_c0,
            stats_spec_c0,
            stats_spec_c0,
            pl.BlockSpec((C0, C1), lambda b, n: (0, 0)),
            stats_spec_c1,
        ],
        out_specs=[
            pl.BlockSpec((1, T, C1), lambda b, n: (b, n, 0)),
            stats_spec_c1,
            stats_spec_c1,
        ],
        out_shape=[
            jax.ShapeDtypeStruct((B, N, C1), jnp.float32),
            jax.ShapeDtypeStruct((1, C1), jnp.float32),
            jax.ShapeDtypeStruct((1, C1), jnp.float32),
        ],
        compiler_params=params,
    )(y0, s0, q0, g0r, be0r, w1t, b1r)

    out = pl.pallas_call(
        functools.partial(_stage_c, inv_m=inv_m),
        grid=(B, NT),
        in_specs=[
            pl.BlockSpec((1, T, C1), lambda b, n: (b, n, 0)),
            stats_spec_c1,
            stats_spec_c1,
            stats_spec_c1,
            stats_spec_c1,
        ],
        out_specs=pl.BlockSpec((1, C1, T), lambda b, n: (b, 0, n)),
        out_shape=jax.ShapeDtypeStruct((B, C1, N), jnp.float32),
        compiler_params=params,
    )(y1, s1, q1, g1r, be1r)

    return out


# speedup vs baseline: 1.0097x; 1.0097x over previous
import functools

import jax
import jax.numpy as jnp
from jax import lax
from jax.experimental import pallas as pl
from jax.experimental.pallas import tpu as pltpu
from jax.experimental.pallas import tpu_sc as plsc

_TILE = 512
_NC = 2
_NS = 16
_LANES = 16


def _stage_a1(xt_ref, y_ref, idx_ref, w_ref, *, S):
    xt = xt_ref[0]
    y = y_ref[0]
    xx = jnp.sum(xt * xt, axis=0, keepdims=True)
    yy = jnp.sum(y * y, axis=1, keepdims=True)
    dt = yy - 2.0 * jnp.dot(y, xt, preferred_element_type=jnp.float32) + xx

    ii = lax.broadcasted_iota(jnp.int32, dt.shape, 0)
    BIG = jnp.float32(3.0e38)
    m1 = jnp.min(dt, axis=0, keepdims=True)
    i1 = jnp.min(jnp.where(dt == m1, ii, S), axis=0, keepdims=True)
    d2 = jnp.where(ii == i1, BIG, dt)
    m2 = jnp.min(d2, axis=0, keepdims=True)
    i2 = jnp.min(jnp.where(d2 == m2, ii, S), axis=0, keepdims=True)
    d3 = jnp.where(ii == i2, BIG, d2)
    m3 = jnp.min(d3, axis=0, keepdims=True)
    i3 = jnp.min(jnp.where(d3 == m3, ii, S), axis=0, keepdims=True)

    mm = jnp.concatenate([m1, m2, m3], axis=0)
    r = 1.0 / (mm + 1e-8)
    w_ref[...] = r / jnp.sum(r, axis=0, keepdims=True)
    idx_ref[...] = (jnp.concatenate([i1, i2, i3], axis=0)
                    + pl.program_id(0) * S)


def _sc_interp(i1, i2, i3, w1, w2, w3, table, BN, D2):
    NW = _NC * _NS
    QW = BN // NW
    Q = 16
    NB = QW // Q
    DCH = D2 // _LANES

    mesh = plsc.VectorSubcoreMesh(core_axis_name="c", subcore_axis_name="s")

    @functools.partial(
        pl.kernel,
        mesh=mesh,
        out_type=jax.ShapeDtypeStruct((BN, D2), jnp.float32),
        scratch_types=[
            pltpu.VMEM((QW,), jnp.int32),
            pltpu.VMEM((QW,), jnp.int32),
            pltpu.VMEM((QW,), jnp.int32),
            pltpu.VMEM((QW,), jnp.float32),
            pltpu.VMEM((QW,), jnp.float32),
            pltpu.VMEM((QW,), jnp.float32),
            pltpu.VMEM((2, Q, D2), jnp.float32),
            pltpu.VMEM((2, Q, D2), jnp.float32),
            pltpu.VMEM((2, Q, D2), jnp.float32),
            pltpu.VMEM((Q, D2), jnp.float32),
            pltpu.SemaphoreType.DMA,
            pltpu.SemaphoreType.DMA,
            pltpu.SemaphoreType.DMA,
            pltpu.SemaphoreType.DMA,
            pltpu.SemaphoreType.DMA,
            pltpu.SemaphoreType.DMA,
        ],
    )
    def gather_interp(i1_hbm, i2_hbm, i3_hbm, w1_hbm, w2_hbm, w3_hbm,
                      table_hbm, out_hbm,
                      i1v, i2v, i3v, w1v, w2v, w3v,
                      r1v, r2v, r3v, outv,
                      s1a, s2a, s3a, s1b, s2b, s3b):
        wid = lax.axis_index("s") * _NC + lax.axis_index("c")
        base = wid * QW
        pltpu.sync_copy(i1_hbm.at[pl.ds(base, QW)], i1v)
        pltpu.sync_copy(i2_hbm.at[pl.ds(base, QW)], i2v)
        pltpu.sync_copy(i3_hbm.at[pl.ds(base, QW)], i3v)
        pltpu.sync_copy(w1_hbm.at[pl.ds(base, QW)], w1v)
        pltpu.sync_copy(w2_hbm.at[pl.ds(base, QW)], w2v)
        pltpu.sync_copy(w3_hbm.at[pl.ds(base, QW)], w3v)

        sems = ((s1a, s2a, s3a), (s1b, s2b, s3b))

        def fire(j, b):
            qb = pl.multiple_of(j * Q, Q)
            pltpu.async_copy(table_hbm.at[i1v[pl.ds(qb, Q)]], r1v.at[b], sems[b][0])
            pltpu.async_copy(table_hbm.at[i2v[pl.ds(qb, Q)]], r2v.at[b], sems[b][1])
            pltpu.async_copy(table_hbm.at[i3v[pl.ds(qb, Q)]], r3v.at[b], sems[b][2])

        fire(0, 0)

        def pair(p, carry):
            for b in range(2):
                j = p * 2 + b
                nb = 1 - b

                @pl.when(j + 1 < NB)
                def _():
                    fire(j + 1, nb)

                for k, rv in enumerate((r1v, r2v, r3v)):
                    pltpu.make_async_copy(
                        table_hbm.at[pl.ds(0, Q)], rv.at[b], sems[b][k]).wait()

                qb = pl.multiple_of(j * Q, Q)
                wa = w1v[pl.ds(qb, Q)]
                wb = w2v[pl.ds(qb, Q)]
                wc = w3v[pl.ds(qb, Q)]
                for q in range(Q):
                    qi = jnp.full((_LANES,), q, jnp.int32)
                    ba = wa.at[qi].get(mode="promise_in_bounds")
                    bb = wb.at[qi].get(mode="promise_in_bounds")
                    bc = wc.at[qi].get(mode="promise_in_bounds")
                    for d in range(DCH):
                        sl = pl.ds(d * _LANES, _LANES)
                        outv[q, sl] = (ba * r1v[b, q, sl] + bb * r2v[b, q, sl]
                                       + bc * r3v[b, q, sl])
                pltpu.sync_copy(outv, out_hbm.at[pl.ds(base + qb, Q)])
            return carry

        lax.fori_loop(0, NB // 2, pair, 0)

    return gather_interp(i1, i2, i3, w1, w2, w3, table)


def _stage_a2(p1_ref, it_ref, w0t_ref, b0_ref, y0_ref, s0_ref, q0_ref):
    cat = jnp.concatenate([p1_ref[0], it_ref[0]], axis=1)
    y0 = jnp.dot(cat, w0t_ref[...], preferred_element_type=jnp.float32) + b0_ref[...]
    y0_ref[0] = y0

    @pl.when((pl.program_id(0) == 0) & (pl.program_id(1) == 0))
    def _():
        s0_ref[...] = jnp.zeros_like(s0_ref)
        q0_ref[...] = jnp.zeros_like(q0_ref)

    s0_ref[...] += jnp.sum(y0, axis=0, keepdims=True)
    q0_ref[...] += jnp.sum(y0 * y0, axis=0, keepdims=True)


def _stage_b(y0_ref, s0_ref, q0_ref, g0_ref, be0_ref, w1t_ref, b1_ref,
             y1_ref, s1_ref, q1_ref, *, inv_m):
    mean = s0_ref[...] * inv_m
    var = q0_ref[...] * inv_m - mean * mean
    scale = g0_ref[...] * lax.rsqrt(var + 1e-5)
    shift = be0_ref[...] - mean * scale
    x1 = jnp.maximum(y0_ref[0] * scale + shift, 0.0)
    y1 = jnp.dot(x1, w1t_ref[...], preferred_element_type=jnp.float32) + b1_ref[...]
    y1_ref[0] = y1

    @pl.when((pl.program_id(0) == 0) & (pl.program_id(1) == 0))
    def _():
        s1_ref[...] = jnp.zeros_like(s1_ref)
        q1_ref[...] = jnp.zeros_like(q1_ref)

    s1_ref[...] += jnp.sum(y1, axis=0, keepdims=True)
    q1_ref[...] += jnp.sum(y1 * y1, axis=0, keepdims=True)


def _stage_c(y1_ref, s1_ref, q1_ref, g1_ref, be1_ref, out_ref, *, inv_m):
    mean = s1_ref[...] * inv_m
    var = q1_ref[...] * inv_m - mean * mean
    scale = g1_ref[...] * lax.rsqrt(var + 1e-5)
    shift = be1_ref[...] - mean * scale
    x2 = jnp.maximum(y1_ref[0] * scale + shift, 0.0)
    out_ref[0] = x2.T


def kernel(xyz1, xyz2, points1, points2, W0, b0, g0, be0, W1, b1, g1, be1):
    B, N, _ = xyz1.shape
    S = xyz2.shape[2]
    D1 = points1.shape[2]
    D2 = points2.shape[1]
    C0 = W0.shape[0]
    C1 = W1.shape[0]
    T = _TILE
    NT = N // T
    BN = B * N
    inv_m = 1.0 / float(BN)

    xyz1t = jnp.transpose(xyz1, (0, 2, 1))
    xyz2t = jnp.transpose(xyz2, (0, 2, 1))
    table = jnp.transpose(points2, (0, 2, 1)).reshape(B * S, D2)
    w0t = W0.T
    w1t = W1.T
    b0r, g0r, be0r = b0.reshape(1, C0), g0.reshape(1, C0), be0.reshape(1, C0)
    b1r, g1r, be1r = b1.reshape(1, C1), g1.reshape(1, C1), be1.reshape(1, C1)

    stats_spec_c0 = pl.BlockSpec((1, C0), lambda b, n: (0, 0))
    stats_spec_c1 = pl.BlockSpec((1, C1), lambda b, n: (0, 0))
    params = pltpu.CompilerParams(dimension_semantics=("arbitrary", "arbitrary"))

    idx3, w3 = pl.pallas_call(
        functools.partial(_stage_a1, S=S),
        grid=(B, NT),
        in_specs=[
            pl.BlockSpec((1, 3, T), lambda b, n: (b, 0, n)),
            pl.BlockSpec((1, S, 3), lambda b, n: (b, 0, 0)),
        ],
        out_specs=[
            pl.BlockSpec((3, T), lambda b, n, NT=NT: (0, b * NT + n)),
            pl.BlockSpec((3, T), lambda b, n, NT=NT: (0, b * NT + n)),
        ],
        out_shape=[
            jax.ShapeDtypeStruct((3, BN), jnp.int32),
            jax.ShapeDtypeStruct((3, BN), jnp.float32),
        ],
        compiler_params=params,
    )(xyz1t, xyz2t)

    interp = _sc_interp(idx3[0], idx3[1], idx3[2], w3[0], w3[1], w3[2],
                        table, BN, D2)
    interp = interp.reshape(B, N, D2)

    y0, s0, q0 = pl.pallas_call(
        _stage_a2,
        grid=(B, NT),
        in_specs=[
            pl.BlockSpec((1, T, D1), lambda b, n: (b, n, 0)),
            pl.BlockSpec((1, T, D2), lambda b, n: (b, n, 0)),
            pl.BlockSpec((D1 + D2, C0), lambda b, n: (0, 0)),
            stats_spec_c0,
        ],
        out_specs=[
            pl.BlockSpec((1, T, C0), lambda b, n: (b, n, 0)),
            stats_spec_c0,
            stats_spec_c0,
        ],
        out_shape=[
            jax.ShapeDtypeStruct((B, N, C0), jnp.float32),
            jax.ShapeDtypeStruct((1, C0), jnp.float32),
            jax.ShapeDtypeStruct((1, C0), jnp.float32),
        ],
        compiler_params=params,
    )(points1, interp, w0t, b0r)

    y1, s1, q1 = pl.pallas_call(
        functools.partial(_stage_b, inv_m=inv_m),
        grid=(B, NT),
        in_specs=[
            pl.BlockSpec((1, T, C0), lambda b, n: (b, n, 0)),
            stats_spec_c0,
            stats_spec_c0,
            stats_spec_c0,
            stats_spec_c0,
            pl.BlockSpec((C0, C1), lambda b, n: (0, 0)),
            stats_spec_c1,
        ],
        out_specs=[
            pl.BlockSpec((1, T, C1), lambda b, n: (b, n, 0)),
            stats_spec_c1,
            stats_spec_c1,
        ],
        out_shape=[
            jax.ShapeDtypeStruct((B, N, C1), jnp.float32),
            jax.ShapeDtypeStruct((1, C1), jnp.float32),
            jax.ShapeDtypeStruct((1, C1), jnp.float32),
        ],
        compiler_params=params,
    )(y0, s0, q0, g0r, be0r, w1t, b1r)

    out = pl.pallas_call(
        functools.partial(_stage_c, inv_m=inv_m),
        grid=(B, NT),
        in_specs=[
            pl.BlockSpec((1, T, C1), lambda b, n: (b, n, 0)),
            stats_spec_c1,
            stats_spec_c1,
            stats_spec_c1,
            stats_spec_c1,
        ],
        out_specs=pl.BlockSpec((1, C1, T), lambda b, n: (b, 0, n)),
        out_shape=jax.ShapeDtypeStruct((B, C1, N), jnp.float32),
        compiler_params=params,
    )(y1, s1, q1, g1r, be1r)

    return out


# speedup vs baseline: 1.0469x; 1.0368x over previous
import functools

import jax
import jax.numpy as jnp
from jax import lax
from jax.experimental import pallas as pl
from jax.experimental.pallas import tpu as pltpu
from jax.experimental.pallas import tpu_sc as plsc

_TILE = 512
_NC = 2
_NS = 16
_LANES = 16


def _stage_a1(xt_ref, y_ref, idx_ref, w_ref, *, S):
    xt = xt_ref[0]
    y = y_ref[0]
    xx = jnp.sum(xt * xt, axis=0, keepdims=True)
    yy = jnp.sum(y * y, axis=1, keepdims=True)
    dt = yy - 2.0 * jnp.dot(y, xt, preferred_element_type=jnp.float32) + xx

    ii = lax.broadcasted_iota(jnp.int32, dt.shape, 0)
    BIG = jnp.float32(3.0e38)
    m1 = jnp.min(dt, axis=0, keepdims=True)
    i1 = jnp.min(jnp.where(dt == m1, ii, S), axis=0, keepdims=True)
    d2 = jnp.where(ii == i1, BIG, dt)
    m2 = jnp.min(d2, axis=0, keepdims=True)
    i2 = jnp.min(jnp.where(d2 == m2, ii, S), axis=0, keepdims=True)
    d3 = jnp.where(ii == i2, BIG, d2)
    m3 = jnp.min(d3, axis=0, keepdims=True)
    i3 = jnp.min(jnp.where(d3 == m3, ii, S), axis=0, keepdims=True)

    mm = jnp.concatenate([m1, m2, m3], axis=0)
    r = 1.0 / (mm + 1e-8)
    w_ref[...] = r / jnp.sum(r, axis=0, keepdims=True)
    idx_ref[...] = (jnp.concatenate([i1, i2, i3], axis=0)
                    + pl.program_id(0) * S)


def _sc_interp(i1, i2, i3, w1, w2, w3, table, BN, D2):
    NW = _NC * _NS
    QW = BN // NW
    Q = 16
    NB = QW // Q
    DCH = D2 // _LANES

    mesh = plsc.VectorSubcoreMesh(core_axis_name="c", subcore_axis_name="s")

    @functools.partial(
        pl.kernel,
        mesh=mesh,
        out_type=jax.ShapeDtypeStruct((BN, D2), jnp.float32),
        scratch_types=[
            pltpu.VMEM((QW,), jnp.int32),
            pltpu.VMEM((QW,), jnp.int32),
            pltpu.VMEM((QW,), jnp.int32),
            pltpu.VMEM((QW,), jnp.float32),
            pltpu.VMEM((QW,), jnp.float32),
            pltpu.VMEM((QW,), jnp.float32),
            pltpu.VMEM((2, Q, D2), jnp.float32),
            pltpu.VMEM((2, Q, D2), jnp.float32),
            pltpu.VMEM((2, Q, D2), jnp.float32),
            pltpu.VMEM((2, Q, D2), jnp.float32),
            pltpu.SemaphoreType.DMA,
            pltpu.SemaphoreType.DMA,
            pltpu.SemaphoreType.DMA,
            pltpu.SemaphoreType.DMA,
            pltpu.SemaphoreType.DMA,
            pltpu.SemaphoreType.DMA,
            pltpu.SemaphoreType.DMA,
            pltpu.SemaphoreType.DMA,
        ],
    )
    def gather_interp(i1_hbm, i2_hbm, i3_hbm, w1_hbm, w2_hbm, w3_hbm,
                      table_hbm, out_hbm,
                      i1v, i2v, i3v, w1v, w2v, w3v,
                      r1v, r2v, r3v, outv,
                      s1a, s2a, s3a, s1b, s2b, s3b, soa, sob):
        wid = lax.axis_index("s") * _NC + lax.axis_index("c")
        base = wid * QW
        pltpu.sync_copy(i1_hbm.at[pl.ds(base, QW)], i1v)
        pltpu.sync_copy(i2_hbm.at[pl.ds(base, QW)], i2v)
        pltpu.sync_copy(i3_hbm.at[pl.ds(base, QW)], i3v)
        pltpu.sync_copy(w1_hbm.at[pl.ds(base, QW)], w1v)
        pltpu.sync_copy(w2_hbm.at[pl.ds(base, QW)], w2v)
        pltpu.sync_copy(w3_hbm.at[pl.ds(base, QW)], w3v)

        sems = ((s1a, s2a, s3a), (s1b, s2b, s3b))
        osems = (soa, sob)

        def fire(j, b):
            qb = pl.multiple_of(j * Q, Q)
            pltpu.async_copy(table_hbm.at[i1v[pl.ds(qb, Q)]], r1v.at[b], sems[b][0])
            pltpu.async_copy(table_hbm.at[i2v[pl.ds(qb, Q)]], r2v.at[b], sems[b][1])
            pltpu.async_copy(table_hbm.at[i3v[pl.ds(qb, Q)]], r3v.at[b], sems[b][2])

        fire(0, 0)

        def pair(p, carry):
            for b in range(2):
                j = p * 2 + b
                nb = 1 - b

                @pl.when(j + 1 < NB)
                def _():
                    fire(j + 1, nb)

                for k, rv in enumerate((r1v, r2v, r3v)):
                    pltpu.make_async_copy(
                        table_hbm.at[pl.ds(0, Q)], rv.at[b], sems[b][k]).wait()

                @pl.when(j >= 2)
                def _():
                    pltpu.make_async_copy(
                        outv.at[b], out_hbm.at[pl.ds(0, Q)], osems[b]).wait()

                qb = pl.multiple_of(j * Q, Q)
                wa = w1v[pl.ds(qb, Q)]
                wb = w2v[pl.ds(qb, Q)]
                wc = w3v[pl.ds(qb, Q)]
                for q in range(Q):
                    qi = jnp.full((_LANES,), q, jnp.int32)
                    ba = wa.at[qi].get(mode="promise_in_bounds")
                    bb = wb.at[qi].get(mode="promise_in_bounds")
                    bc = wc.at[qi].get(mode="promise_in_bounds")
                    for d in range(DCH):
                        sl = pl.ds(d * _LANES, _LANES)
                        outv[b, q, sl] = (ba * r1v[b, q, sl] + bb * r2v[b, q, sl]
                                          + bc * r3v[b, q, sl])
                pltpu.async_copy(outv.at[b], out_hbm.at[pl.ds(base + qb, Q)],
                                 osems[b])
            return carry

        lax.fori_loop(0, NB // 2, pair, 0)
        for b in range(2):
            pltpu.make_async_copy(
                outv.at[b], out_hbm.at[pl.ds(0, Q)], osems[b]).wait()

    return gather_interp(i1, i2, i3, w1, w2, w3, table)


def _stage_a2(p1_ref, it_ref, w0t_ref, b0_ref, y0_ref, s0_ref, q0_ref):
    cat = jnp.concatenate([p1_ref[0], it_ref[0]], axis=1)
    y0 = jnp.dot(cat, w0t_ref[...], preferred_element_type=jnp.float32) + b0_ref[...]
    y0_ref[0] = y0

    @pl.when((pl.program_id(0) == 0) & (pl.program_id(1) == 0))
    def _():
        s0_ref[...] = jnp.zeros_like(s0_ref)
        q0_ref[...] = jnp.zeros_like(q0_ref)

    s0_ref[...] += jnp.sum(y0, axis=0, keepdims=True)
    q0_ref[...] += jnp.sum(y0 * y0, axis=0, keepdims=True)


def _stage_b(y0_ref, s0_ref, q0_ref, g0_ref, be0_ref, w1t_ref, b1_ref,
             y1_ref, s1_ref, q1_ref, *, inv_m):
    mean = s0_ref[...] * inv_m
    var = q0_ref[...] * inv_m - mean * mean
    scale = g0_ref[...] * lax.rsqrt(var + 1e-5)
    shift = be0_ref[...] - mean * scale
    x1 = jnp.maximum(y0_ref[0] * scale + shift, 0.0)
    y1 = jnp.dot(x1, w1t_ref[...], preferred_element_type=jnp.float32) + b1_ref[...]
    y1_ref[0] = y1

    @pl.when((pl.program_id(0) == 0) & (pl.program_id(1) == 0))
    def _():
        s1_ref[...] = jnp.zeros_like(s1_ref)
        q1_ref[...] = jnp.zeros_like(q1_ref)

    s1_ref[...] += jnp.sum(y1, axis=0, keepdims=True)
    q1_ref[...] += jnp.sum(y1 * y1, axis=0, keepdims=True)


def _stage_c(y1_ref, s1_ref, q1_ref, g1_ref, be1_ref, out_ref, *, inv_m):
    mean = s1_ref[...] * inv_m
    var = q1_ref[...] * inv_m - mean * mean
    scale = g1_ref[...] * lax.rsqrt(var + 1e-5)
    shift = be1_ref[...] - mean * scale
    x2 = jnp.maximum(y1_ref[0] * scale + shift, 0.0)
    out_ref[0] = x2.T


def kernel(xyz1, xyz2, points1, points2, W0, b0, g0, be0, W1, b1, g1, be1):
    B, N, _ = xyz1.shape
    S = xyz2.shape[2]
    D1 = points1.shape[2]
    D2 = points2.shape[1]
    C0 = W0.shape[0]
    C1 = W1.shape[0]
    T = _TILE
    NT = N // T
    BN = B * N
    inv_m = 1.0 / float(BN)

    xyz1t = jnp.transpose(xyz1, (0, 2, 1))
    xyz2t = jnp.transpose(xyz2, (0, 2, 1))
    table = jnp.transpose(points2, (0, 2, 1)).reshape(B * S, D2)
    w0t = W0.T
    w1t = W1.T
    b0r, g0r, be0r = b0.reshape(1, C0), g0.reshape(1, C0), be0.reshape(1, C0)
    b1r, g1r, be1r = b1.reshape(1, C1), g1.reshape(1, C1), be1.reshape(1, C1)

    stats_spec_c0 = pl.BlockSpec((1, C0), lambda b, n: (0, 0))
    stats_spec_c1 = pl.BlockSpec((1, C1), lambda b, n: (0, 0))
    params = pltpu.CompilerParams(dimension_semantics=("arbitrary", "arbitrary"))

    idx3, w3 = pl.pallas_call(
        functools.partial(_stage_a1, S=S),
        grid=(B, NT),
        in_specs=[
            pl.BlockSpec((1, 3, T), lambda b, n: (b, 0, n)),
            pl.BlockSpec((1, S, 3), lambda b, n: (b, 0, 0)),
        ],
        out_specs=[
            pl.BlockSpec((3, T), lambda b, n, NT=NT: (0, b * NT + n)),
            pl.BlockSpec((3, T), lambda b, n, NT=NT: (0, b * NT + n)),
        ],
        out_shape=[
            jax.ShapeDtypeStruct((3, BN), jnp.int32),
            jax.ShapeDtypeStruct((3, BN), jnp.float32),
        ],
        compiler_params=params,
    )(xyz1t, xyz2t)

    interp = _sc_interp(idx3[0], idx3[1], idx3[2], w3[0], w3[1], w3[2],
                        table, BN, D2)
    interp = interp.reshape(B, N, D2)

    y0, s0, q0 = pl.pallas_call(
        _stage_a2,
        grid=(B, NT),
        in_specs=[
            pl.BlockSpec((1, T, D1), lambda b, n: (b, n, 0)),
            pl.BlockSpec((1, T, D2), lambda b, n: (b, n, 0)),
            pl.BlockSpec((D1 + D2, C0), lambda b, n: (0, 0)),
            stats_spec_c0,
        ],
        out_specs=[
            pl.BlockSpec((1, T, C0), lambda b, n: (b, n, 0)),
            stats_spec_c0,
            stats_spec_c0,
        ],
        out_shape=[
            jax.ShapeDtypeStruct((B, N, C0), jnp.float32),
            jax.ShapeDtypeStruct((1, C0), jnp.float32),
            jax.ShapeDtypeStruct((1, C0), jnp.float32),
        ],
        compiler_params=params,
    )(points1, interp, w0t, b0r)

    y1, s1, q1 = pl.pallas_call(
        functools.partial(_stage_b, inv_m=inv_m),
        grid=(B, NT),
        in_specs=[
            pl.BlockSpec((1, T, C0), lambda b, n: (b, n, 0)),
            stats_spec_c0,
            stats_spec_c0,
            stats_spec_c0,
            stats_spec_c0,
            pl.BlockSpec((C0, C1), lambda b, n: (0, 0)),
            stats_spec_c1,
        ],
        out_specs=[
            pl.BlockSpec((1, T, C1), lambda b, n: (b, n, 0)),
            stats_spec_c1,
            stats_spec_c1,
        ],
        out_shape=[
            jax.ShapeDtypeStruct((B, N, C1), jnp.float32),
            jax.ShapeDtypeStruct((1, C1), jnp.float32),
            jax.ShapeDtypeStruct((1, C1), jnp.float32),
        ],
        compiler_params=params,
    )(y0, s0, q0, g0r, be0r, w1t, b1r)

    out = pl.pallas_call(
        functools.partial(_stage_c, inv_m=inv_m),
        grid=(B, NT),
        in_specs=[
            pl.BlockSpec((1, T, C1), lambda b, n: (b, n, 0)),
            stats_spec_c1,
            stats_spec_c1,
            stats_spec_c1,
            stats_spec_c1,
        ],
        out_specs=pl.BlockSpec((1, C1, T), lambda b, n: (b, 0, n)),
        out_shape=jax.ShapeDtypeStruct((B, C1, N), jnp.float32),
        compiler_params=params,
    )(y1, s1, q1, g1r, be1r)

    return out
